# baseline (speedup 1.0000x reference)
import jax
import jax.numpy as jnp
from jax import lax
from jax.experimental import pallas as pl
from jax.experimental.pallas import tpu as pltpu
from jax.experimental.pallas import tpu_sc as plsc

N_NODES = 10000
D = 128
E = 320000
NC = 2
NS = 16
NW = NC * NS
EPW = E // NW
C = 80
NCHUNK = EPW // C


def _dot_body(h_hbm, ei_hbm, out_hbm,
              a_src, a_dst, u0, v0, u1, v1, ob0, ob1,
              su0, sv0, su1, sv1):
    wid = lax.axis_index("s") * NC + lax.axis_index("c")
    base0 = wid * EPW
    pltpu.sync_copy(ei_hbm.at[0, pl.ds(base0, EPW)], a_src)
    pltpu.sync_copy(ei_hbm.at[1, pl.ds(base0, EPW)], a_dst)

    lane = lax.iota(jnp.int32, 16)
    perm = {s: lane ^ s for s in (8, 4, 2, 1)}
    mask = {s: (lane & s) == 0 for s in (8, 4, 2, 1)}
    BITREV = [0, 8, 4, 12, 2, 10, 6, 14, 1, 9, 5, 13, 3, 11, 7, 15]

    def merge(x, y, s):
        m = mask[s]
        a = jnp.where(m, x, y)
        b = jnp.where(m, y, x)
        return a + b.at[perm[s]].get(mode="promise_in_bounds")

    def fire(i, u, v, su, sv):
        pltpu.async_copy(h_hbm.at[a_src.at[pl.ds(i * C, C)]], u, su)
        pltpu.async_copy(h_hbm.at[a_dst.at[pl.ds(i * C, C)]], v, sv)

    def wait(i, u, v, su, sv):
        pltpu.make_async_copy(h_hbm.at[a_src.at[pl.ds(i * C, C)]], u, su).wait()
        pltpu.make_async_copy(h_hbm.at[a_dst.at[pl.ds(i * C, C)]], v, sv).wait()

    def compute(i, u_rows, v_rows, out_buf):
        def edge_acc(e):
            prods = []
            for j in range(4):
                wu = u_rows[e, pl.ds(16 * j, 16)]
                wv = v_rows[e, pl.ds(16 * j, 16)]
                prods.append(plsc.bitcast(wu, jnp.bfloat16)
                             * plsc.bitcast(wv, jnp.bfloat16))
            acc = None
            for j in (0, 2):
                pw = plsc.bitcast(prods[j] + prods[j + 1], jnp.int32)
                pa = plsc.bitcast(lax.shift_left(pw, 16), jnp.float32)
                pb = plsc.bitcast(
                    jnp.bitwise_and(pw, jnp.int32(-65536)), jnp.float32)
                t = pa + pb
                acc = t if acc is None else acc + t
            return acc

        def group_body(g, gcarry):
            e0 = g * 16
            stack = []
            for idx in range(16):
                node = (0, edge_acc(e0 + BITREV[idx]))
                while stack and stack[-1][0] == node[0]:
                    lvl, x = stack.pop()
                    node = (lvl + 1, merge(x, node[1], (8, 4, 2, 1)[lvl]))
                stack.append(node)
            out_buf[pl.ds(e0, 16)] = stack[0][1]
            return gcarry

        for g in range(C // 16):
            group_body(g, 0)
        pltpu.sync_copy(out_buf, out_hbm.at[pl.ds(base0 + i * C, C)])

    fire(0, u0, v0, su0, sv0)

    def body(j, carry):
        c0 = 2 * j
        fire(c0 + 1, u1, v1, su1, sv1)
        wait(c0, u0, v0, su0, sv0)
        compute(c0, u0, v0, ob0)
        fire(c0 + 2, u0, v0, su0, sv0)
        wait(c0 + 1, u1, v1, su1, sv1)
        compute(c0 + 1, u1, v1, ob1)
        return carry

    lax.fori_loop(0, (NCHUNK - 1) // 2, body, 0)
    wait(NCHUNK - 1, u0, v0, su0, sv0)
    compute(NCHUNK - 1, u0, v0, ob0)


def kernel(h, edge_index):
    w = lax.bitcast_convert_type(h, jnp.uint32)
    b = (w + jnp.uint32(0x7FFF) + ((w >> 16) & jnp.uint32(1))) >> 16
    h = lax.bitcast_convert_type(
        b[:, : D // 2] | (b[:, D // 2:] << 16), jnp.int32)
    mesh = plsc.VectorSubcoreMesh(core_axis_name="c", subcore_axis_name="s")
    f = pl.kernel(
        _dot_body,
        out_type=jax.ShapeDtypeStruct((E,), jnp.float32),
        mesh=mesh,
        compiler_params=pltpu.CompilerParams(
            needs_layout_passes=False, use_tc_tiling_on_sc=False),
        scratch_types=[
            pltpu.VMEM((EPW,), jnp.int32),
            pltpu.VMEM((EPW,), jnp.int32),
            pltpu.VMEM((C, D // 2), jnp.int32),
            pltpu.VMEM((C, D // 2), jnp.int32),
            pltpu.VMEM((C, D // 2), jnp.int32),
            pltpu.VMEM((C, D // 2), jnp.int32),
            pltpu.VMEM((C,), jnp.float32),
            pltpu.VMEM((C,), jnp.float32),
            pltpu.SemaphoreType.DMA,
            pltpu.SemaphoreType.DMA,
            pltpu.SemaphoreType.DMA,
            pltpu.SemaphoreType.DMA,
        ],
    )
    return f(h, edge_index)


# speedup vs baseline: 1.7567x; 1.7567x over previous
import jax
import jax.numpy as jnp
from jax import lax
from jax.experimental import pallas as pl
from jax.experimental.pallas import tpu as pltpu
from jax.experimental.pallas import tpu_sc as plsc

N_NODES = 10000
D = 128
E = 320000
NC = 2
NS = 16
NW = NC * NS
EPW = E // NW
C = 80
NCHUNK = EPW // C


def _dot_body(h_hbm, ei_hbm, out_hbm,
              a_src, a_dst, u0, v0, u1, v1, ob0, ob1,
              su0, sv0, su1, sv1):
    wid = lax.axis_index("s") * NC + lax.axis_index("c")
    base0 = wid * EPW
    pltpu.sync_copy(ei_hbm.at[0, pl.ds(base0, EPW)], a_src)
    pltpu.sync_copy(ei_hbm.at[1, pl.ds(base0, EPW)], a_dst)

    lane = lax.iota(jnp.int32, 16)
    perm = {s: lane ^ s for s in (8, 4, 2, 1)}
    mask = {s: (lane & s) == 0 for s in (8, 4, 2, 1)}
    BITREV = [0, 8, 4, 12, 2, 10, 6, 14, 1, 9, 5, 13, 3, 11, 7, 15]

    def merge(x, y, s):
        m = mask[s]
        a = jnp.where(m, x, y)
        b = jnp.where(m, y, x)
        return a + b.at[perm[s]].get(mode="promise_in_bounds")

    def fire(i, u, v, su, sv):
        pltpu.async_copy(h_hbm.at[a_src.at[pl.ds(i * C, C)]], u, su)
        pltpu.async_copy(h_hbm.at[a_dst.at[pl.ds(i * C, C)]], v, sv)

    def wait(i, u, v, su, sv):
        pltpu.make_async_copy(h_hbm.at[a_src.at[pl.ds(i * C, C)]], u, su).wait()
        pltpu.make_async_copy(h_hbm.at[a_dst.at[pl.ds(i * C, C)]], v, sv).wait()

    def compute(i, u_rows, v_rows, out_buf):
        def edge_acc(e):
            prods = []
            for j in range(4):
                wu = u_rows[e, pl.ds(16 * j, 16)]
                wv = v_rows[e, pl.ds(16 * j, 16)]
                prods.append(plsc.bitcast(wu, jnp.bfloat16)
                             * plsc.bitcast(wv, jnp.bfloat16))
            acc = None
            for j in (0, 2):
                pw = plsc.bitcast(prods[j] + prods[j + 1], jnp.int32)
                pa = plsc.bitcast(lax.shift_left(pw, 16), jnp.float32)
                pb = plsc.bitcast(
                    jnp.bitwise_and(pw, jnp.int32(-65536)), jnp.float32)
                t = pa + pb
                acc = t if acc is None else acc + t
            return acc

        def edge_accs(g):
            return tuple(edge_acc(g * 16 + BITREV[idx]) for idx in range(16))

        def tree(accs):
            stack = []
            for a in accs:
                node = (0, a)
                while stack and stack[-1][0] == node[0]:
                    lvl, x = stack.pop()
                    node = (lvl + 1, merge(x, node[1], (8, 4, 2, 1)[lvl]))
                stack.append(node)
            return stack[0][1]

        def group_body(g, accs):
            new = edge_accs(g)
            out_buf[pl.ds((g - 1) * 16, 16)] = tree(accs)
            return new

        last = lax.fori_loop(1, C // 16, group_body, edge_accs(0))
        out_buf[pl.ds(C - 16, 16)] = tree(last)
        pltpu.sync_copy(out_buf, out_hbm.at[pl.ds(base0 + i * C, C)])

    fire(0, u0, v0, su0, sv0)

    def body(j, carry):
        c0 = 2 * j
        fire(c0 + 1, u1, v1, su1, sv1)
        wait(c0, u0, v0, su0, sv0)
        compute(c0, u0, v0, ob0)
        fire(c0 + 2, u0, v0, su0, sv0)
        wait(c0 + 1, u1, v1, su1, sv1)
        compute(c0 + 1, u1, v1, ob1)
        return carry

    lax.fori_loop(0, (NCHUNK - 1) // 2, body, 0)
    wait(NCHUNK - 1, u0, v0, su0, sv0)
    compute(NCHUNK - 1, u0, v0, ob0)


def kernel(h, edge_index):
    w = lax.bitcast_convert_type(h, jnp.uint32)
    b = (w + jnp.uint32(0x7FFF) + ((w >> 16) & jnp.uint32(1))) >> 16
    h = lax.bitcast_convert_type(
        b[:, : D // 2] | (b[:, D // 2:] << 16), jnp.int32)
    mesh = plsc.VectorSubcoreMesh(core_axis_name="c", subcore_axis_name="s")
    f = pl.kernel(
        _dot_body,
        out_type=jax.ShapeDtypeStruct((E,), jnp.float32),
        mesh=mesh,
        compiler_params=pltpu.CompilerParams(
            needs_layout_passes=False, use_tc_tiling_on_sc=False),
        scratch_types=[
            pltpu.VMEM((EPW,), jnp.int32),
            pltpu.VMEM((EPW,), jnp.int32),
            pltpu.VMEM((C, D // 2), jnp.int32),
            pltpu.VMEM((C, D // 2), jnp.int32),
            pltpu.VMEM((C, D // 2), jnp.int32),
            pltpu.VMEM((C, D // 2), jnp.int32),
            pltpu.VMEM((C,), jnp.float32),
            pltpu.VMEM((C,), jnp.float32),
            pltpu.SemaphoreType.DMA,
            pltpu.SemaphoreType.DMA,
            pltpu.SemaphoreType.DMA,
            pltpu.SemaphoreType.DMA,
        ],
    )
    return f(h, edge_index)
